# baseline (device time: 16466 ns/iter reference)
import jax
import jax.numpy as jnp
from jax import lax
from jax.experimental import pallas as pl
from jax.experimental.pallas import tpu as pltpu

N_DEV = 16
N_PAIR = 8
_DP = (4, 3, 5, 2, 6, 1, 7, 0)


def kernel(x, w_mat):
    m_per, k = x.shape
    n = w_mat.shape[1]
    n_per = n // N_DEV
    m_out = m_per * N_DEV

    def body(order_ref, x_ref, w_ref, out_ref, xb_ref, y_blk_ref,
             send_sems, recv_sems):
        g = pl.program_id(0)
        my = lax.axis_index("i")
        barrier = pltpu.get_barrier_semaphore()

        @pl.when(g == 0)
        def _():
            for d in range(1, N_DEV):
                j = lax.rem(my + d, N_DEV)
                pl.semaphore_signal(
                    barrier, inc=1,
                    device_id=(j,), device_id_type=pl.DeviceIdType.MESH,
                )
            pl.semaphore_wait(barrier, N_DEV - 1)
            xb_ref[:, :] = x_ref[:, :].astype(jnp.bfloat16)

        pair = order_ref[g]
        wb = w_ref[:, :].astype(jnp.bfloat16)
        yb = lax.dot(xb_ref[:, :], wb, preferred_element_type=jnp.float32)
        yb = (yb * jax.nn.sigmoid(yb)).astype(jnp.bfloat16)
        y_blk_ref[2 * pair, :, :] = yb[:, :n_per]
        y_blk_ref[2 * pair + 1, :, :] = yb[:, n_per:]

        for t in range(2):
            j = 2 * pair + t

            @pl.when(j == my)
            def _():
                out_ref[pl.ds(my * m_per, m_per), :] = y_blk_ref[j, :, :]

            @pl.when(j != my)
            def _():
                rdma = pltpu.make_async_remote_copy(
                    src_ref=y_blk_ref.at[j],
                    dst_ref=out_ref.at[pl.ds(my * m_per, m_per), :],
                    send_sem=send_sems.at[2 * g + t],
                    recv_sem=recv_sems.at[my],
                    device_id=(j,),
                    device_id_type=pl.DeviceIdType.MESH,
                )
                rdma.start()

        @pl.when(g == N_PAIR - 1)
        def _():
            for d in range(1, N_DEV):
                i = lax.rem(my - d + N_DEV, N_DEV)
                recv = pltpu.make_async_remote_copy(
                    src_ref=y_blk_ref.at[0],
                    dst_ref=out_ref.at[pl.ds(i * m_per, m_per), :],
                    send_sem=send_sems.at[0],
                    recv_sem=recv_sems.at[i],
                    device_id=(i,),
                    device_id_type=pl.DeviceIdType.MESH,
                )
                recv.wait_recv()

            for s in range(2 * N_PAIR):
                blk = 2 * order_ref[s // 2] + (s % 2)

                @pl.when(blk != my)
                def _():
                    snd = pltpu.make_async_remote_copy(
                        src_ref=y_blk_ref.at[0],
                        dst_ref=out_ref.at[pl.ds(0, m_per), :],
                        send_sem=send_sems.at[s],
                        recv_sem=recv_sems.at[0],
                        device_id=(0,),
                        device_id_type=pl.DeviceIdType.MESH,
                    )
                    snd.wait_send()

    my = lax.axis_index("i")
    order = lax.rem(my // 2 + jnp.array(_DP, jnp.int32), N_PAIR)

    grid_spec = pltpu.PrefetchScalarGridSpec(
        num_scalar_prefetch=1,
        grid=(N_PAIR,),
        in_specs=[
            pl.BlockSpec((m_per, k), lambda g, order_ref: (0, 0)),
            pl.BlockSpec((k, 2 * n_per), lambda g, order_ref: (0, order_ref[g])),
        ],
        out_specs=pl.BlockSpec((m_out, n_per), lambda g, order_ref: (0, 0)),
        scratch_shapes=[
            pltpu.VMEM((m_per, k), jnp.bfloat16),
            pltpu.VMEM((N_DEV, m_per, n_per), jnp.bfloat16),
            pltpu.SemaphoreType.DMA((2 * N_PAIR,)),
            pltpu.SemaphoreType.DMA((N_DEV,)),
        ],
    )

    return pl.pallas_call(
        body,
        grid_spec=grid_spec,
        out_shape=jax.ShapeDtypeStruct((m_out, n_per), jnp.bfloat16),
        compiler_params=pltpu.CompilerParams(collective_id=0),
    )(order, x, w_mat)


# device time: 14754 ns/iter; 1.1160x vs baseline; 1.1160x over previous
import jax
import jax.numpy as jnp
from jax import lax
from jax.experimental import pallas as pl
from jax.experimental.pallas import tpu as pltpu

N_DEV = 16
N_PAIR = 8
_DP = (4, 3, 5, 2, 6, 1, 7, 0)


def kernel(x, w_mat):
    m_per, k = x.shape
    n = w_mat.shape[1]
    n_per = n // N_DEV
    m_out = m_per * N_DEV
    slab = 2 * n_per

    def body(x_ref, w_ref, out_ref, xb_ref, wv_ref, y_blk_ref,
             copy_sems, send_sems, recv_sems):
        my = lax.axis_index("i")
        my_pair = my // 2
        barrier = pltpu.get_barrier_semaphore()

        for d in range(1, N_DEV):
            j = lax.rem(my + d, N_DEV)
            pl.semaphore_signal(
                barrier, inc=1,
                device_id=(j,), device_id_type=pl.DeviceIdType.MESH,
            )

        copies = []
        for g in range(N_PAIR):
            pair = lax.rem(my_pair + _DP[g], N_PAIR)
            cp = pltpu.make_async_copy(
                w_ref.at[:, pl.ds(pair * slab, slab)],
                wv_ref.at[g],
                copy_sems.at[g],
            )
            cp.start()
            copies.append(cp)

        xb_ref[:, :] = x_ref[:, :].astype(jnp.bfloat16)

        pl.semaphore_wait(barrier, N_DEV - 1)

        for g in range(N_PAIR):
            pair = lax.rem(my_pair + _DP[g], N_PAIR)
            copies[g].wait()
            wb = wv_ref[g, :, :].astype(jnp.bfloat16)
            yb = lax.dot(xb_ref[:, :], wb, preferred_element_type=jnp.float32)
            yb = (yb * jax.nn.sigmoid(yb)).astype(jnp.bfloat16)
            y_blk_ref[2 * pair, :, :] = yb[:, :n_per]
            y_blk_ref[2 * pair + 1, :, :] = yb[:, n_per:]

            for t in range(2):
                j = 2 * pair + t

                @pl.when(j == my)
                def _():
                    out_ref[pl.ds(my * m_per, m_per), :] = y_blk_ref[j, :, :]

                @pl.when(j != my)
                def _():
                    rdma = pltpu.make_async_remote_copy(
                        src_ref=y_blk_ref.at[j],
                        dst_ref=out_ref.at[pl.ds(my * m_per, m_per), :],
                        send_sem=send_sems.at[2 * g + t],
                        recv_sem=recv_sems.at[my],
                        device_id=(j,),
                        device_id_type=pl.DeviceIdType.MESH,
                    )
                    rdma.start()

        for d in range(1, N_DEV):
            i = lax.rem(my - d + N_DEV, N_DEV)
            recv = pltpu.make_async_remote_copy(
                src_ref=y_blk_ref.at[0],
                dst_ref=out_ref.at[pl.ds(i * m_per, m_per), :],
                send_sem=send_sems.at[0],
                recv_sem=recv_sems.at[i],
                device_id=(i,),
                device_id_type=pl.DeviceIdType.MESH,
            )
            recv.wait_recv()

        for g in range(N_PAIR):
            pair = lax.rem(my_pair + _DP[g], N_PAIR)
            for t in range(2):
                blk = 2 * pair + t

                @pl.when(blk != my)
                def _():
                    snd = pltpu.make_async_remote_copy(
                        src_ref=y_blk_ref.at[0],
                        dst_ref=out_ref.at[pl.ds(0, m_per), :],
                        send_sem=send_sems.at[2 * g + t],
                        recv_sem=recv_sems.at[0],
                        device_id=(0,),
                        device_id_type=pl.DeviceIdType.MESH,
                    )
                    snd.wait_send()

    return pl.pallas_call(
        body,
        out_shape=jax.ShapeDtypeStruct((m_out, n_per), jnp.bfloat16),
        in_specs=[
            pl.BlockSpec(memory_space=pltpu.VMEM),
            pl.BlockSpec(memory_space=pl.MemorySpace.ANY),
        ],
        out_specs=pl.BlockSpec(memory_space=pltpu.VMEM),
        scratch_shapes=[
            pltpu.VMEM((m_per, k), jnp.bfloat16),
            pltpu.VMEM((N_PAIR, k, slab), jnp.float32),
            pltpu.VMEM((N_DEV, m_per, n_per), jnp.bfloat16),
            pltpu.SemaphoreType.DMA((N_PAIR,)),
            pltpu.SemaphoreType.DMA((2 * N_PAIR,)),
            pltpu.SemaphoreType.DMA((N_DEV,)),
        ],
        compiler_params=pltpu.CompilerParams(collective_id=0),
    )(x, w_mat)


# device time: 14712 ns/iter; 1.1192x vs baseline; 1.0029x over previous
import jax
import jax.numpy as jnp
from jax import lax
from jax.experimental import pallas as pl
from jax.experimental.pallas import tpu as pltpu

N_DEV = 16
N_PAIR = 8
_DP = (4, 3, 5, 2, 6, 1, 7, 0)


def kernel(x, w_mat):
    m_per, k = x.shape
    n = w_mat.shape[1]
    n_per = n // N_DEV
    m_out = m_per * N_DEV
    slab = 2 * n_per

    def body(x_ref, w_ref, out_ref, xf_ref, xb_ref, wv_ref, y_blk_ref,
             xcopy_sem, copy_sems, send_sems, recv_sems):
        my = lax.axis_index("i")
        my_pair = my // 2
        barrier = pltpu.get_barrier_semaphore()

        for d in range(1, N_DEV):
            j = lax.rem(my + d, N_DEV)
            pl.semaphore_signal(
                barrier, inc=1,
                device_id=(j,), device_id_type=pl.DeviceIdType.MESH,
            )

        xcp = pltpu.make_async_copy(x_ref, xf_ref, xcopy_sem)
        xcp.start()
        copies = []
        for g in range(N_PAIR):
            pair = lax.rem(my_pair + _DP[g], N_PAIR)
            cp = pltpu.make_async_copy(
                w_ref.at[:, pl.ds(pair * slab, slab)],
                wv_ref.at[g],
                copy_sems.at[g],
            )
            cp.start()
            copies.append(cp)

        xcp.wait()
        xb_ref[:, :] = xf_ref[:, :].astype(jnp.bfloat16)

        pl.semaphore_wait(barrier, N_DEV - 1)

        for g in range(N_PAIR):
            pair = lax.rem(my_pair + _DP[g], N_PAIR)
            copies[g].wait()
            wb = wv_ref[g, :, :].astype(jnp.bfloat16)
            yb = lax.dot(xb_ref[:, :], wb, preferred_element_type=jnp.float32)
            yb = (yb * jax.nn.sigmoid(yb)).astype(jnp.bfloat16)
            y_blk_ref[2 * pair, :, :] = yb[:, :n_per]
            y_blk_ref[2 * pair + 1, :, :] = yb[:, n_per:]

            for t in range(2):
                j = 2 * pair + t

                @pl.when(j == my)
                def _():
                    out_ref[pl.ds(my * m_per, m_per), :] = y_blk_ref[j, :, :]

                @pl.when(j != my)
                def _():
                    rdma = pltpu.make_async_remote_copy(
                        src_ref=y_blk_ref.at[j],
                        dst_ref=out_ref.at[pl.ds(my * m_per, m_per), :],
                        send_sem=send_sems.at[2 * g + t],
                        recv_sem=recv_sems.at[my],
                        device_id=(j,),
                        device_id_type=pl.DeviceIdType.MESH,
                    )
                    rdma.start()

        for d in range(1, N_DEV):
            i = lax.rem(my - d + N_DEV, N_DEV)
            recv = pltpu.make_async_remote_copy(
                src_ref=y_blk_ref.at[0],
                dst_ref=out_ref.at[pl.ds(i * m_per, m_per), :],
                send_sem=send_sems.at[0],
                recv_sem=recv_sems.at[i],
                device_id=(i,),
                device_id_type=pl.DeviceIdType.MESH,
            )
            recv.wait_recv()

        for g in range(N_PAIR):
            pair = lax.rem(my_pair + _DP[g], N_PAIR)
            for t in range(2):
                blk = 2 * pair + t

                @pl.when(blk != my)
                def _():
                    snd = pltpu.make_async_remote_copy(
                        src_ref=y_blk_ref.at[0],
                        dst_ref=out_ref.at[pl.ds(0, m_per), :],
                        send_sem=send_sems.at[2 * g + t],
                        recv_sem=recv_sems.at[0],
                        device_id=(0,),
                        device_id_type=pl.DeviceIdType.MESH,
                    )
                    snd.wait_send()

    return pl.pallas_call(
        body,
        out_shape=jax.ShapeDtypeStruct((m_out, n_per), jnp.bfloat16),
        in_specs=[
            pl.BlockSpec(memory_space=pltpu.MemorySpace.HBM),
            pl.BlockSpec(memory_space=pltpu.MemorySpace.HBM),
        ],
        out_specs=pl.BlockSpec(memory_space=pltpu.VMEM),
        scratch_shapes=[
            pltpu.VMEM((m_per, k), jnp.float32),
            pltpu.VMEM((m_per, k), jnp.bfloat16),
            pltpu.VMEM((N_PAIR, k, slab), jnp.float32),
            pltpu.VMEM((N_DEV, m_per, n_per), jnp.bfloat16),
            pltpu.SemaphoreType.DMA,
            pltpu.SemaphoreType.DMA((N_PAIR,)),
            pltpu.SemaphoreType.DMA((2 * N_PAIR,)),
            pltpu.SemaphoreType.DMA((N_DEV,)),
        ],
        compiler_params=pltpu.CompilerParams(collective_id=0),
    )(x, w_mat)


# device time: 13877 ns/iter; 1.1866x vs baseline; 1.0602x over previous
import jax
import jax.numpy as jnp
from jax import lax
from jax.experimental import pallas as pl
from jax.experimental.pallas import tpu as pltpu

N_DEV = 16
N_PAIR = 8
_DP = (4, 3, 5, 2, 6, 1, 7, 0)


def kernel(x, w_mat):
    m_per, k = x.shape
    n = w_mat.shape[1]
    n_per = n // N_DEV
    m_out = m_per * N_DEV
    slab = 2 * n_per

    def body(x_ref, w_ref, out_ref, xb_ref, y_blk_ref, send_sems, recv_sems):
        my = lax.axis_index("i")
        my_pair = my // 2

        barrier = pltpu.get_barrier_semaphore()
        pl.semaphore_signal(barrier, 1)
        pl.semaphore_wait(barrier, 1)

        xb_ref[:, :] = x_ref[:, :].astype(jnp.bfloat16)

        for g in range(N_PAIR):
            pair = lax.rem(my_pair + _DP[g], N_PAIR)
            wb = w_ref[:, pl.ds(pair * slab, slab)].astype(jnp.bfloat16)
            yb = lax.dot(xb_ref[:, :], wb, preferred_element_type=jnp.float32)
            yb = (yb * jax.nn.sigmoid(yb)).astype(jnp.bfloat16)
            y_blk_ref[2 * pair, :, :] = yb[:, :n_per]
            y_blk_ref[2 * pair + 1, :, :] = yb[:, n_per:]

            for t in range(2):
                j = 2 * pair + t

                @pl.when(j == my)
                def _():
                    out_ref[pl.ds(my * m_per, m_per), :] = y_blk_ref[j, :, :]

                @pl.when(j != my)
                def _():
                    rdma = pltpu.make_async_remote_copy(
                        src_ref=y_blk_ref.at[j],
                        dst_ref=out_ref.at[pl.ds(my * m_per, m_per), :],
                        send_sem=send_sems.at[2 * g + t],
                        recv_sem=recv_sems.at[my],
                        device_id=(j,),
                        device_id_type=pl.DeviceIdType.MESH,
                    )
                    rdma.start()

        for d in range(1, N_DEV):
            i = lax.rem(my - d + N_DEV, N_DEV)
            recv = pltpu.make_async_remote_copy(
                src_ref=y_blk_ref.at[0],
                dst_ref=out_ref.at[pl.ds(i * m_per, m_per), :],
                send_sem=send_sems.at[0],
                recv_sem=recv_sems.at[i],
                device_id=(i,),
                device_id_type=pl.DeviceIdType.MESH,
            )
            recv.wait_recv()

        for g in range(N_PAIR):
            pair = lax.rem(my_pair + _DP[g], N_PAIR)
            for t in range(2):
                blk = 2 * pair + t

                @pl.when(blk != my)
                def _():
                    snd = pltpu.make_async_remote_copy(
                        src_ref=y_blk_ref.at[0],
                        dst_ref=out_ref.at[pl.ds(0, m_per), :],
                        send_sem=send_sems.at[2 * g + t],
                        recv_sem=recv_sems.at[0],
                        device_id=(0,),
                        device_id_type=pl.DeviceIdType.MESH,
                    )
                    snd.wait_send()

    return pl.pallas_call(
        body,
        out_shape=jax.ShapeDtypeStruct((m_out, n_per), jnp.bfloat16),
        in_specs=[
            pl.BlockSpec(memory_space=pltpu.VMEM),
            pl.BlockSpec(memory_space=pltpu.VMEM),
        ],
        out_specs=pl.BlockSpec(memory_space=pltpu.VMEM),
        scratch_shapes=[
            pltpu.VMEM((m_per, k), jnp.bfloat16),
            pltpu.VMEM((N_DEV, m_per, n_per), jnp.bfloat16),
            pltpu.SemaphoreType.DMA((2 * N_PAIR,)),
            pltpu.SemaphoreType.DMA((N_DEV,)),
        ],
        compiler_params=pltpu.CompilerParams(collective_id=0),
    )(x, w_mat)


# device time: 13463 ns/iter; 1.2231x vs baseline; 1.0308x over previous
import jax
import jax.numpy as jnp
from jax import lax
from jax.experimental import pallas as pl
from jax.experimental.pallas import tpu as pltpu

N_DEV = 16


def kernel(x, w_mat):
    m_per, k = x.shape
    n = w_mat.shape[1]
    n_per = n // N_DEV
    m_out = m_per * N_DEV

    def body(x_ref, w_ref, out_ref, y_blk_ref, send_sems, recv_sems):
        my = lax.axis_index("i")

        barrier = pltpu.get_barrier_semaphore()
        for d in range(1, N_DEV):
            j = lax.rem(my + d, N_DEV)
            pl.semaphore_signal(
                barrier, inc=1,
                device_id=(j,), device_id_type=pl.DeviceIdType.MESH,
            )

        xb = x_ref[:, :].astype(jnp.bfloat16)
        wb = w_ref[:, :].astype(jnp.bfloat16)
        y = lax.dot(xb, wb, preferred_element_type=jnp.float32)
        y = (y * jax.nn.sigmoid(y)).astype(jnp.bfloat16)

        for j in range(N_DEV):
            y_blk_ref[j, :, :] = y[:, j * n_per:(j + 1) * n_per]

        out_ref[pl.ds(my * m_per, m_per), :] = y_blk_ref[my, :, :]

        pl.semaphore_wait(barrier, N_DEV - 1)

        for d in range(1, N_DEV):
            j = lax.rem(my + d, N_DEV)
            rdma = pltpu.make_async_remote_copy(
                src_ref=y_blk_ref.at[j],
                dst_ref=out_ref.at[pl.ds(my * m_per, m_per), :],
                send_sem=send_sems.at[d],
                recv_sem=recv_sems.at[my],
                device_id=(j,),
                device_id_type=pl.DeviceIdType.MESH,
            )
            rdma.start()

        for d in range(1, N_DEV):
            i = lax.rem(my - d + N_DEV, N_DEV)
            recv = pltpu.make_async_remote_copy(
                src_ref=y_blk_ref.at[0],
                dst_ref=out_ref.at[pl.ds(i * m_per, m_per), :],
                send_sem=send_sems.at[0],
                recv_sem=recv_sems.at[i],
                device_id=(i,),
                device_id_type=pl.DeviceIdType.MESH,
            )
            recv.wait_recv()

        for d in range(1, N_DEV):
            snd = pltpu.make_async_remote_copy(
                src_ref=y_blk_ref.at[0],
                dst_ref=out_ref.at[pl.ds(0, m_per), :],
                send_sem=send_sems.at[d],
                recv_sem=recv_sems.at[0],
                device_id=(0,),
                device_id_type=pl.DeviceIdType.MESH,
            )
            snd.wait_send()

    return pl.pallas_call(
        body,
        out_shape=jax.ShapeDtypeStruct((m_out, n_per), jnp.bfloat16),
        in_specs=[
            pl.BlockSpec(memory_space=pltpu.VMEM),
            pl.BlockSpec(memory_space=pltpu.VMEM),
        ],
        out_specs=pl.BlockSpec(memory_space=pltpu.VMEM),
        scratch_shapes=[
            pltpu.VMEM((N_DEV, m_per, n_per), jnp.bfloat16),
            pltpu.SemaphoreType.DMA((N_DEV,)),
            pltpu.SemaphoreType.DMA((N_DEV,)),
        ],
        compiler_params=pltpu.CompilerParams(collective_id=0),
    )(x, w_mat)
